# baseline (speedup 1.0000x reference)
import functools
import math

import jax
import jax.numpy as jnp
from jax import lax
from jax.experimental import pallas as pl
from jax.experimental.pallas import tpu as pltpu
from jax.experimental.pallas import tpu_sc as plsc

_N = 262144
_L = 16
_NC = 2
_NS = 16
_NW = _NC * _NS
_RW = _N // _NW
_C = 2048
_NCHUNK = _RW // _C
_LOG_SPLIT = math.log(0.8 * 2)
_GRAD_THRESHOLD = 0.5
_MAX_THRESHOLD = 0.1 * 5.0


def _densify_body(xyz_h, scal_h, rot_h, grads_h, noise_h,
                  oxyz_h, oscal_h, orot_h,
                  xyz_v, scal_v, rot_v, grads_v, n0_v, n1_v,
                  ox0_v, ox1_v, os_v, or_v):
  cid = lax.axis_index("c")
  sid = lax.axis_index("s")
  wid = sid * _NC + cid
  base = wid * _RW
  iota = lax.iota(jnp.int32, _L)
  i3 = iota * 3
  i4 = iota * 4
  i3_0, i3_1, i3_2 = i3, i3 + 1, i3 + 2
  i4_0, i4_1, i4_2, i4_3 = i4, i4 + 1, i4 + 2, i4 + 3

  def chunk(k, carry):
    off = base + k * _C
    pltpu.sync_copy(xyz_h.at[pl.ds(3 * off, 3 * _C)], xyz_v)
    pltpu.sync_copy(scal_h.at[pl.ds(3 * off, 3 * _C)], scal_v)
    pltpu.sync_copy(rot_h.at[pl.ds(4 * off, 4 * _C)], rot_v)
    pltpu.sync_copy(grads_h.at[pl.ds(off, _C)], grads_v)
    pltpu.sync_copy(noise_h.at[pl.ds(3 * off, 3 * _C)], n0_v)
    pltpu.sync_copy(noise_h.at[pl.ds(3 * (_N + off), 3 * _C)], n1_v)

    def step(i, scarry):
      b3 = i * (3 * _L)
      b4 = i * (4 * _L)
      r3_0 = i3_0 + b3
      r3_1 = i3_1 + b3
      r3_2 = i3_2 + b3
      r4_0 = i4_0 + b4
      r4_1 = i4_1 + b4
      r4_2 = i4_2 + b4
      r4_3 = i4_3 + b4
      r1 = iota + i * _L

      s0 = plsc.load_gather(scal_v, [r3_0])
      s1 = plsc.load_gather(scal_v, [r3_1])
      s2 = plsc.load_gather(scal_v, [r3_2])
      e0 = jnp.exp(s0)
      e1 = jnp.exp(s1)
      e2 = jnp.exp(s2)
      g = plsc.load_gather(grads_v, [r1])
      smax = jnp.maximum(jnp.maximum(e0, e1), e2)
      sel = (g >= _GRAD_THRESHOLD) & (smax > _MAX_THRESHOLD)
      m = jnp.where(sel, jnp.float32(1.0), jnp.float32(0.0))

      q0 = plsc.load_gather(rot_v, [r4_0])
      q1 = plsc.load_gather(rot_v, [r4_1])
      q2 = plsc.load_gather(rot_v, [r4_2])
      q3 = plsc.load_gather(rot_v, [r4_3])
      q11 = q1 * q1
      q22 = q2 * q2
      q33 = q3 * q3
      ss = q0 * q0 + q11 + q22 + q33
      t = jnp.float32(2.0) / ss
      q12 = q1 * q2
      q13 = q1 * q3
      q23 = q2 * q3
      q01 = q0 * q1
      q02 = q0 * q2
      q03 = q0 * q3
      r00 = jnp.float32(1.0) - t * (q22 + q33)
      r01 = t * (q12 - q03)
      r02 = t * (q13 + q02)
      r10 = t * (q12 + q03)
      r11 = jnp.float32(1.0) - t * (q11 + q33)
      r12 = t * (q23 - q01)
      r20 = t * (q13 - q02)
      r21 = t * (q23 + q01)
      r22 = jnp.float32(1.0) - t * (q11 + q22)

      px = plsc.load_gather(xyz_v, [r3_0])
      py = plsc.load_gather(xyz_v, [r3_1])
      pz = plsc.load_gather(xyz_v, [r3_2])

      for n_v, ox_v in ((n0_v, ox0_v), (n1_v, ox1_v)):
        a0 = plsc.load_gather(n_v, [r3_0]) * e0
        a1 = plsc.load_gather(n_v, [r3_1]) * e1
        a2 = plsc.load_gather(n_v, [r3_2]) * e2
        ox = (r00 * a0 + r01 * a1 + r02 * a2 + px) * m
        oy = (r10 * a0 + r11 * a1 + r12 * a2 + py) * m
        oz = (r20 * a0 + r21 * a1 + r22 * a2 + pz) * m
        plsc.store_scatter(ox_v, [r3_0], ox)
        plsc.store_scatter(ox_v, [r3_1], oy)
        plsc.store_scatter(ox_v, [r3_2], oz)

      plsc.store_scatter(os_v, [r3_0], (s0 - _LOG_SPLIT) * m)
      plsc.store_scatter(os_v, [r3_1], (s1 - _LOG_SPLIT) * m)
      plsc.store_scatter(os_v, [r3_2], (s2 - _LOG_SPLIT) * m)
      plsc.store_scatter(or_v, [r4_0], q0 * m)
      plsc.store_scatter(or_v, [r4_1], q1 * m)
      plsc.store_scatter(or_v, [r4_2], q2 * m)
      plsc.store_scatter(or_v, [r4_3], q3 * m)
      return scarry

    lax.fori_loop(0, _C // _L, step, 0)

    pltpu.sync_copy(ox0_v, oxyz_h.at[pl.ds(3 * off, 3 * _C)])
    pltpu.sync_copy(ox1_v, oxyz_h.at[pl.ds(3 * (_N + off), 3 * _C)])
    pltpu.sync_copy(os_v, oscal_h.at[pl.ds(3 * off, 3 * _C)])
    pltpu.sync_copy(os_v, oscal_h.at[pl.ds(3 * (_N + off), 3 * _C)])
    pltpu.sync_copy(or_v, orot_h.at[pl.ds(4 * off, 4 * _C)])
    pltpu.sync_copy(or_v, orot_h.at[pl.ds(4 * (_N + off), 4 * _C)])
    return carry

  lax.fori_loop(0, _NCHUNK, chunk, 0)


_densify = functools.partial(
    pl.kernel,
    out_type=(
        jax.ShapeDtypeStruct((2 * _N * 3,), jnp.float32),
        jax.ShapeDtypeStruct((2 * _N * 3,), jnp.float32),
        jax.ShapeDtypeStruct((2 * _N * 4,), jnp.float32),
    ),
    mesh=plsc.VectorSubcoreMesh(core_axis_name="c", subcore_axis_name="s"),
    compiler_params=pltpu.CompilerParams(needs_layout_passes=False),
    scratch_types=[
        pltpu.VMEM((3 * _C,), jnp.float32),
        pltpu.VMEM((3 * _C,), jnp.float32),
        pltpu.VMEM((4 * _C,), jnp.float32),
        pltpu.VMEM((_C,), jnp.float32),
        pltpu.VMEM((3 * _C,), jnp.float32),
        pltpu.VMEM((3 * _C,), jnp.float32),
        pltpu.VMEM((3 * _C,), jnp.float32),
        pltpu.VMEM((3 * _C,), jnp.float32),
        pltpu.VMEM((3 * _C,), jnp.float32),
        pltpu.VMEM((4 * _C,), jnp.float32),
    ],
)(_densify_body)


def kernel(xyz, scaling, rotation, grads, noise):
  nxyz, nscal, nrot = _densify(
      xyz.reshape(-1), scaling.reshape(-1), rotation.reshape(-1),
      grads.reshape(-1), noise.reshape(-1))
  return (nxyz.reshape(2 * _N, 3),
          nscal.reshape(2 * _N, 3),
          nrot.reshape(2 * _N, 4))


# speedup vs baseline: 1.0117x; 1.0117x over previous
import functools
import math

import jax
import jax.numpy as jnp
from jax import lax
from jax.experimental import pallas as pl
from jax.experimental.pallas import tpu as pltpu
from jax.experimental.pallas import tpu_sc as plsc

_N = 262144
_L = 16
_NC = 2
_NS = 16
_NW = _NC * _NS
_RW = _N // _NW
_C = 2048
_NCHUNK = _RW // _C
_NBUF = 2
_LOG_SPLIT = math.log(0.8 * 2)
_GRAD_THRESHOLD = 0.5
_MAX_THRESHOLD = 0.1 * 5.0


def _densify_body(xyz_h, scal_h, rot_h, grads_h, noise_h,
                  oxyz_h, oscal_h, orot_h, *scratch):
  in_bufs = [scratch[6 * b:6 * b + 6] for b in range(_NBUF)]
  out_bufs = [scratch[12 + 4 * b:12 + 4 * b + 4] for b in range(_NBUF)]
  in_sems = scratch[20:22]
  out_sems = scratch[22:24]

  cid = lax.axis_index("c")
  sid = lax.axis_index("s")
  wid = sid * _NC + cid
  base = wid * _RW
  iota = lax.iota(jnp.int32, _L)
  i3 = iota * 3
  i4 = iota * 4
  i3_0, i3_1, i3_2 = i3, i3 + 1, i3 + 2
  i4_0, i4_1, i4_2, i4_3 = i4, i4 + 1, i4 + 2, i4 + 3

  def start_in(k, b):
    off = base + k * _C
    xyz_v, scal_v, rot_v, grads_v, n0_v, n1_v = in_bufs[b]
    sem = in_sems[b]
    return [
        pltpu.async_copy(xyz_h.at[pl.ds(3 * off, 3 * _C)], xyz_v, sem),
        pltpu.async_copy(scal_h.at[pl.ds(3 * off, 3 * _C)], scal_v, sem),
        pltpu.async_copy(rot_h.at[pl.ds(4 * off, 4 * _C)], rot_v, sem),
        pltpu.async_copy(grads_h.at[pl.ds(off, _C)], grads_v, sem),
        pltpu.async_copy(noise_h.at[pl.ds(3 * off, 3 * _C)], n0_v, sem),
        pltpu.async_copy(noise_h.at[pl.ds(3 * (_N + off), 3 * _C)], n1_v,
                         sem),
    ]

  def start_out(k, b):
    off = base + k * _C
    ox0_v, ox1_v, os_v, or_v = out_bufs[b]
    sem = out_sems[b]
    return [
        pltpu.async_copy(ox0_v, oxyz_h.at[pl.ds(3 * off, 3 * _C)], sem),
        pltpu.async_copy(ox1_v, oxyz_h.at[pl.ds(3 * (_N + off), 3 * _C)],
                         sem),
        pltpu.async_copy(os_v, oscal_h.at[pl.ds(3 * off, 3 * _C)], sem),
        pltpu.async_copy(os_v, oscal_h.at[pl.ds(3 * (_N + off), 3 * _C)],
                         sem),
        pltpu.async_copy(or_v, orot_h.at[pl.ds(4 * off, 4 * _C)], sem),
        pltpu.async_copy(or_v, orot_h.at[pl.ds(4 * (_N + off), 4 * _C)],
                         sem),
    ]

  def compute(b):
    xyz_v, scal_v, rot_v, grads_v, n0_v, n1_v = in_bufs[b]
    ox0_v, ox1_v, os_v, or_v = out_bufs[b]

    def step(i, scarry):
      b3 = i * (3 * _L)
      b4 = i * (4 * _L)
      r3_0 = i3_0 + b3
      r3_1 = i3_1 + b3
      r3_2 = i3_2 + b3
      r4_0 = i4_0 + b4
      r4_1 = i4_1 + b4
      r4_2 = i4_2 + b4
      r4_3 = i4_3 + b4
      r1 = iota + i * _L

      s0 = plsc.load_gather(scal_v, [r3_0])
      s1 = plsc.load_gather(scal_v, [r3_1])
      s2 = plsc.load_gather(scal_v, [r3_2])
      e0 = jnp.exp(s0)
      e1 = jnp.exp(s1)
      e2 = jnp.exp(s2)
      g = plsc.load_gather(grads_v, [r1])
      smax = jnp.maximum(jnp.maximum(e0, e1), e2)
      sel = (g >= _GRAD_THRESHOLD) & (smax > _MAX_THRESHOLD)
      m = jnp.where(sel, jnp.float32(1.0), jnp.float32(0.0))

      q0 = plsc.load_gather(rot_v, [r4_0])
      q1 = plsc.load_gather(rot_v, [r4_1])
      q2 = plsc.load_gather(rot_v, [r4_2])
      q3 = plsc.load_gather(rot_v, [r4_3])
      q11 = q1 * q1
      q22 = q2 * q2
      q33 = q3 * q3
      ss = q0 * q0 + q11 + q22 + q33
      t = jnp.float32(2.0) / ss
      q12 = q1 * q2
      q13 = q1 * q3
      q23 = q2 * q3
      q01 = q0 * q1
      q02 = q0 * q2
      q03 = q0 * q3
      r00 = jnp.float32(1.0) - t * (q22 + q33)
      r01 = t * (q12 - q03)
      r02 = t * (q13 + q02)
      r10 = t * (q12 + q03)
      r11 = jnp.float32(1.0) - t * (q11 + q33)
      r12 = t * (q23 - q01)
      r20 = t * (q13 - q02)
      r21 = t * (q23 + q01)
      r22 = jnp.float32(1.0) - t * (q11 + q22)

      px = plsc.load_gather(xyz_v, [r3_0])
      py = plsc.load_gather(xyz_v, [r3_1])
      pz = plsc.load_gather(xyz_v, [r3_2])

      for n_v, ox_v in ((n0_v, ox0_v), (n1_v, ox1_v)):
        a0 = plsc.load_gather(n_v, [r3_0]) * e0
        a1 = plsc.load_gather(n_v, [r3_1]) * e1
        a2 = plsc.load_gather(n_v, [r3_2]) * e2
        ox = (r00 * a0 + r01 * a1 + r02 * a2 + px) * m
        oy = (r10 * a0 + r11 * a1 + r12 * a2 + py) * m
        oz = (r20 * a0 + r21 * a1 + r22 * a2 + pz) * m
        plsc.store_scatter(ox_v, [r3_0], ox)
        plsc.store_scatter(ox_v, [r3_1], oy)
        plsc.store_scatter(ox_v, [r3_2], oz)

      plsc.store_scatter(os_v, [r3_0], (s0 - _LOG_SPLIT) * m)
      plsc.store_scatter(os_v, [r3_1], (s1 - _LOG_SPLIT) * m)
      plsc.store_scatter(os_v, [r3_2], (s2 - _LOG_SPLIT) * m)
      plsc.store_scatter(or_v, [r4_0], q0 * m)
      plsc.store_scatter(or_v, [r4_1], q1 * m)
      plsc.store_scatter(or_v, [r4_2], q2 * m)
      plsc.store_scatter(or_v, [r4_3], q3 * m)
      return scarry

    lax.fori_loop(0, _C // _L, step, 0)

  pending_in = {0: start_in(0, 0)}
  pending_out = {}
  for k in range(_NCHUNK):
    b = k % _NBUF
    if k + 1 < _NCHUNK:
      pending_in[k + 1] = start_in(k + 1, (k + 1) % _NBUF)
    for h in pending_in.pop(k):
      h.wait()
    if k - _NBUF in pending_out:
      for h in pending_out.pop(k - _NBUF):
        h.wait()
    compute(b)
    pending_out[k] = start_out(k, b)
  for k in sorted(pending_out):
    for h in pending_out[k]:
      h.wait()


_densify = functools.partial(
    pl.kernel,
    out_type=(
        jax.ShapeDtypeStruct((2 * _N * 3,), jnp.float32),
        jax.ShapeDtypeStruct((2 * _N * 3,), jnp.float32),
        jax.ShapeDtypeStruct((2 * _N * 4,), jnp.float32),
    ),
    mesh=plsc.VectorSubcoreMesh(core_axis_name="c", subcore_axis_name="s"),
    compiler_params=pltpu.CompilerParams(needs_layout_passes=False),
    scratch_types=(
        [t for _ in range(_NBUF) for t in (
            pltpu.VMEM((3 * _C,), jnp.float32),
            pltpu.VMEM((3 * _C,), jnp.float32),
            pltpu.VMEM((4 * _C,), jnp.float32),
            pltpu.VMEM((_C,), jnp.float32),
            pltpu.VMEM((3 * _C,), jnp.float32),
            pltpu.VMEM((3 * _C,), jnp.float32),
        )] +
        [t for _ in range(_NBUF) for t in (
            pltpu.VMEM((3 * _C,), jnp.float32),
            pltpu.VMEM((3 * _C,), jnp.float32),
            pltpu.VMEM((3 * _C,), jnp.float32),
            pltpu.VMEM((4 * _C,), jnp.float32),
        )] +
        [pltpu.SemaphoreType.DMA] * 4
    ),
)(_densify_body)


def kernel(xyz, scaling, rotation, grads, noise):
  nxyz, nscal, nrot = _densify(
      xyz.reshape(-1), scaling.reshape(-1), rotation.reshape(-1),
      grads.reshape(-1), noise.reshape(-1))
  return (nxyz.reshape(2 * _N, 3),
          nscal.reshape(2 * _N, 3),
          nrot.reshape(2 * _N, 4))


# speedup vs baseline: 19.5356x; 19.3099x over previous
import functools
import math

import jax
import jax.numpy as jnp
from jax import lax
from jax.experimental import pallas as pl
from jax.experimental.pallas import tpu as pltpu
from jax.experimental.pallas import tpu_sc as plsc

_N = 262144
_M = 2 * _N
_L = 16
_NC = 2
_NS = 16
_NW = _NC * _NS
_RW = _N // _NW
_C = 2048
_NCHUNK = _RW // _C
_NBUF = 2
_LOG_SPLIT = math.log(0.8 * 2)
_GRAD_THRESHOLD = 0.5
_MAX_THRESHOLD = 0.1 * 5.0


def _densify_body(xyz_h, scal_h, rot_h, grads_h, noise_h,
                  oxyz_h, oscal_h, orot_h, *scratch):
  in_bufs = [scratch[6 * b:6 * b + 6] for b in range(_NBUF)]
  out_bufs = [scratch[12 + 4 * b:12 + 4 * b + 4] for b in range(_NBUF)]
  in_sems = scratch[20:22]
  out_sems = scratch[22:24]

  cid = lax.axis_index("c")
  sid = lax.axis_index("s")
  wid = sid * _NC + cid
  base = wid * _RW

  def start_in(k, b):
    off = base + k * _C
    xyz_v, scal_v, rot_v, grads_v, n0_v, n1_v = in_bufs[b]
    sem = in_sems[b]
    h = []
    for c in range(3):
      h.append(pltpu.async_copy(xyz_h.at[pl.ds(c * _N + off, _C)],
                                xyz_v.at[pl.ds(c * _C, _C)], sem))
      h.append(pltpu.async_copy(scal_h.at[pl.ds(c * _N + off, _C)],
                                scal_v.at[pl.ds(c * _C, _C)], sem))
      h.append(pltpu.async_copy(noise_h.at[pl.ds(c * _M + off, _C)],
                                n0_v.at[pl.ds(c * _C, _C)], sem))
      h.append(pltpu.async_copy(noise_h.at[pl.ds(c * _M + _N + off, _C)],
                                n1_v.at[pl.ds(c * _C, _C)], sem))
    for c in range(4):
      h.append(pltpu.async_copy(rot_h.at[pl.ds(c * _N + off, _C)],
                                rot_v.at[pl.ds(c * _C, _C)], sem))
    h.append(pltpu.async_copy(grads_h.at[pl.ds(off, _C)], grads_v, sem))
    return h

  def start_out(k, b):
    off = base + k * _C
    ox0_v, ox1_v, os_v, or_v = out_bufs[b]
    sem = out_sems[b]
    h = []
    for c in range(3):
      h.append(pltpu.async_copy(ox0_v.at[pl.ds(c * _C, _C)],
                                oxyz_h.at[pl.ds(c * _M + off, _C)], sem))
      h.append(pltpu.async_copy(ox1_v.at[pl.ds(c * _C, _C)],
                                oxyz_h.at[pl.ds(c * _M + _N + off, _C)], sem))
      h.append(pltpu.async_copy(os_v.at[pl.ds(c * _C, _C)],
                                oscal_h.at[pl.ds(c * _M + off, _C)], sem))
      h.append(pltpu.async_copy(os_v.at[pl.ds(c * _C, _C)],
                                oscal_h.at[pl.ds(c * _M + _N + off, _C)], sem))
    for c in range(4):
      h.append(pltpu.async_copy(or_v.at[pl.ds(c * _C, _C)],
                                orot_h.at[pl.ds(c * _M + off, _C)], sem))
      h.append(pltpu.async_copy(or_v.at[pl.ds(c * _C, _C)],
                                orot_h.at[pl.ds(c * _M + _N + off, _C)], sem))
    return h

  def compute(b):
    xyz_v, scal_v, rot_v, grads_v, n0_v, n1_v = in_bufs[b]
    ox0_v, ox1_v, os_v, or_v = out_bufs[b]

    def step(i, scarry):
      j = i * _L

      s0 = scal_v[pl.ds(j, _L)]
      s1 = scal_v[pl.ds(_C + j, _L)]
      s2 = scal_v[pl.ds(2 * _C + j, _L)]
      e0 = jnp.exp(s0)
      e1 = jnp.exp(s1)
      e2 = jnp.exp(s2)
      g = grads_v[pl.ds(j, _L)]
      smax = jnp.maximum(jnp.maximum(e0, e1), e2)
      sel = (g >= _GRAD_THRESHOLD) & (smax > _MAX_THRESHOLD)
      m = jnp.where(sel, jnp.float32(1.0), jnp.float32(0.0))

      q0 = rot_v[pl.ds(j, _L)]
      q1 = rot_v[pl.ds(_C + j, _L)]
      q2 = rot_v[pl.ds(2 * _C + j, _L)]
      q3 = rot_v[pl.ds(3 * _C + j, _L)]
      q11 = q1 * q1
      q22 = q2 * q2
      q33 = q3 * q3
      ss = q0 * q0 + q11 + q22 + q33
      t = jnp.float32(2.0) / ss
      q12 = q1 * q2
      q13 = q1 * q3
      q23 = q2 * q3
      q01 = q0 * q1
      q02 = q0 * q2
      q03 = q0 * q3
      r00 = jnp.float32(1.0) - t * (q22 + q33)
      r01 = t * (q12 - q03)
      r02 = t * (q13 + q02)
      r10 = t * (q12 + q03)
      r11 = jnp.float32(1.0) - t * (q11 + q33)
      r12 = t * (q23 - q01)
      r20 = t * (q13 - q02)
      r21 = t * (q23 + q01)
      r22 = jnp.float32(1.0) - t * (q11 + q22)

      px = xyz_v[pl.ds(j, _L)]
      py = xyz_v[pl.ds(_C + j, _L)]
      pz = xyz_v[pl.ds(2 * _C + j, _L)]

      for n_v, ox_v in ((n0_v, ox0_v), (n1_v, ox1_v)):
        a0 = n_v[pl.ds(j, _L)] * e0
        a1 = n_v[pl.ds(_C + j, _L)] * e1
        a2 = n_v[pl.ds(2 * _C + j, _L)] * e2
        ox_v[pl.ds(j, _L)] = (r00 * a0 + r01 * a1 + r02 * a2 + px) * m
        ox_v[pl.ds(_C + j, _L)] = (r10 * a0 + r11 * a1 + r12 * a2 + py) * m
        ox_v[pl.ds(2 * _C + j, _L)] = (r20 * a0 + r21 * a1 + r22 * a2 + pz) * m

      os_v[pl.ds(j, _L)] = (s0 - _LOG_SPLIT) * m
      os_v[pl.ds(_C + j, _L)] = (s1 - _LOG_SPLIT) * m
      os_v[pl.ds(2 * _C + j, _L)] = (s2 - _LOG_SPLIT) * m
      or_v[pl.ds(j, _L)] = q0 * m
      or_v[pl.ds(_C + j, _L)] = q1 * m
      or_v[pl.ds(2 * _C + j, _L)] = q2 * m
      or_v[pl.ds(3 * _C + j, _L)] = q3 * m
      return scarry

    lax.fori_loop(0, _C // _L, step, 0)

  pending_in = {0: start_in(0, 0)}
  pending_out = {}
  for k in range(_NCHUNK):
    b = k % _NBUF
    if k + 1 < _NCHUNK:
      pending_in[k + 1] = start_in(k + 1, (k + 1) % _NBUF)
    for h in pending_in.pop(k):
      h.wait()
    if k - _NBUF in pending_out:
      for h in pending_out.pop(k - _NBUF):
        h.wait()
    compute(b)
    pending_out[k] = start_out(k, b)
  for k in sorted(pending_out):
    for h in pending_out[k]:
      h.wait()


_densify = functools.partial(
    pl.kernel,
    out_type=(
        jax.ShapeDtypeStruct((3 * _M,), jnp.float32),
        jax.ShapeDtypeStruct((3 * _M,), jnp.float32),
        jax.ShapeDtypeStruct((4 * _M,), jnp.float32),
    ),
    mesh=plsc.VectorSubcoreMesh(core_axis_name="c", subcore_axis_name="s"),
    compiler_params=pltpu.CompilerParams(
        needs_layout_passes=False, use_tc_tiling_on_sc=False),
    scratch_types=(
        [t for _ in range(_NBUF) for t in (
            pltpu.VMEM((3 * _C,), jnp.float32),
            pltpu.VMEM((3 * _C,), jnp.float32),
            pltpu.VMEM((4 * _C,), jnp.float32),
            pltpu.VMEM((_C,), jnp.float32),
            pltpu.VMEM((3 * _C,), jnp.float32),
            pltpu.VMEM((3 * _C,), jnp.float32),
        )] +
        [t for _ in range(_NBUF) for t in (
            pltpu.VMEM((3 * _C,), jnp.float32),
            pltpu.VMEM((3 * _C,), jnp.float32),
            pltpu.VMEM((3 * _C,), jnp.float32),
            pltpu.VMEM((4 * _C,), jnp.float32),
        )] +
        [pltpu.SemaphoreType.DMA] * 4
    ),
)(_densify_body)


def kernel(xyz, scaling, rotation, grads, noise):
  nxyz, nscal, nrot = _densify(
      xyz.T.reshape(-1), scaling.T.reshape(-1), rotation.T.reshape(-1),
      grads.reshape(-1), noise.T.reshape(-1))
  return (nxyz.reshape(3, _M).T,
          nscal.reshape(3, _M).T,
          nrot.reshape(4, _M).T)


# speedup vs baseline: 24.1954x; 1.2385x over previous
import functools
import math

import jax
import jax.numpy as jnp
from jax import lax
from jax.experimental import pallas as pl
from jax.experimental.pallas import tpu as pltpu
from jax.experimental.pallas import tpu_sc as plsc

_N = 262144
_M = 2 * _N
_B = 128
_L = 16
_NC = 2
_NS = 16
_NW = _NC * _NS
_RW = _N // _NW
_C = 1024
_NCHUNK = _RW // _C
_NBUF = 2
_LOG_SPLIT = math.log(0.8 * 2)
_GRAD_THRESHOLD = 0.5
_MAX_THRESHOLD = 0.1 * 5.0


def _densify_body(xyz_h, scal_h, rot_h, grads_h, noise_h,
                  oxyz_h, oscal_h, orot_h, *scratch):
  in_bufs = [scratch[6 * b:6 * b + 6] for b in range(_NBUF)]
  out_bufs = [scratch[12 + 4 * b:12 + 4 * b + 4] for b in range(_NBUF)]
  in_sems = scratch[20:22]
  out_sems = scratch[22:24]

  cid = lax.axis_index("c")
  sid = lax.axis_index("s")
  wid = sid * _NC + cid
  base = wid * _RW

  def start_in(k, b):
    off = base + k * _C
    xyz_v, scal_v, rot_v, grads_v, n0_v, n1_v = in_bufs[b]
    sem = in_sems[b]
    h = []
    for c in range(3):
      h.append(pltpu.async_copy(xyz_h.at[pl.ds(c * _N + off, _C)],
                                xyz_v.at[pl.ds(c * _C, _C)], sem))
      h.append(pltpu.async_copy(scal_h.at[pl.ds(c * _N + off, _C)],
                                scal_v.at[pl.ds(c * _C, _C)], sem))
      h.append(pltpu.async_copy(noise_h.at[pl.ds(c * _M + off, _C)],
                                n0_v.at[pl.ds(c * _C, _C)], sem))
      h.append(pltpu.async_copy(noise_h.at[pl.ds(c * _M + _N + off, _C)],
                                n1_v.at[pl.ds(c * _C, _C)], sem))
    h.append(pltpu.async_copy(rot_h.at[pl.ds(4 * off, 4 * _C)], rot_v, sem))
    h.append(pltpu.async_copy(grads_h.at[pl.ds(off, _C)], grads_v, sem))
    return h

  def start_out(k, b):
    off = base + k * _C
    ox0_v, ox1_v, os_v, or_v = out_bufs[b]
    sem = out_sems[b]
    return [
        pltpu.async_copy(ox0_v, oxyz_h.at[pl.ds(4 * off, 4 * _C)], sem),
        pltpu.async_copy(ox1_v, oxyz_h.at[pl.ds(4 * (_N + off), 4 * _C)],
                         sem),
        pltpu.async_copy(os_v, oscal_h.at[pl.ds(4 * off, 4 * _C)], sem),
        pltpu.async_copy(os_v, oscal_h.at[pl.ds(4 * (_N + off), 4 * _C)],
                         sem),
        pltpu.async_copy(or_v, orot_h.at[pl.ds(4 * off, 4 * _C)], sem),
        pltpu.async_copy(or_v, orot_h.at[pl.ds(4 * (_N + off), 4 * _C)],
                         sem),
    ]

  def compute(b):
    xyz_v, scal_v, rot_v, grads_v, n0_v, n1_v = in_bufs[b]
    ox0_v, ox1_v, os_v, or_v = out_bufs[b]

    def step(i, scarry):
      j = i * _L
      ji = (i // (_B // _L)) * (4 * _B) + (i % (_B // _L)) * _L

      s0 = scal_v[pl.ds(j, _L)]
      s1 = scal_v[pl.ds(_C + j, _L)]
      s2 = scal_v[pl.ds(2 * _C + j, _L)]
      e0 = jnp.exp(s0)
      e1 = jnp.exp(s1)
      e2 = jnp.exp(s2)
      g = grads_v[pl.ds(j, _L)]
      smax = jnp.maximum(jnp.maximum(e0, e1), e2)
      sel = (g >= _GRAD_THRESHOLD) & (smax > _MAX_THRESHOLD)
      m = jnp.where(sel, jnp.float32(1.0), jnp.float32(0.0))

      q0 = rot_v[pl.ds(ji, _L)]
      q1 = rot_v[pl.ds(ji + _B, _L)]
      q2 = rot_v[pl.ds(ji + 2 * _B, _L)]
      q3 = rot_v[pl.ds(ji + 3 * _B, _L)]
      q11 = q1 * q1
      q22 = q2 * q2
      q33 = q3 * q3
      ss = q0 * q0 + q11 + q22 + q33
      t = jnp.float32(2.0) / ss
      q12 = q1 * q2
      q13 = q1 * q3
      q23 = q2 * q3
      q01 = q0 * q1
      q02 = q0 * q2
      q03 = q0 * q3
      r00 = jnp.float32(1.0) - t * (q22 + q33)
      r01 = t * (q12 - q03)
      r02 = t * (q13 + q02)
      r10 = t * (q12 + q03)
      r11 = jnp.float32(1.0) - t * (q11 + q33)
      r12 = t * (q23 - q01)
      r20 = t * (q13 - q02)
      r21 = t * (q23 + q01)
      r22 = jnp.float32(1.0) - t * (q11 + q22)

      px = xyz_v[pl.ds(j, _L)]
      py = xyz_v[pl.ds(_C + j, _L)]
      pz = xyz_v[pl.ds(2 * _C + j, _L)]

      for n_v, ox_v in ((n0_v, ox0_v), (n1_v, ox1_v)):
        a0 = n_v[pl.ds(j, _L)] * e0
        a1 = n_v[pl.ds(_C + j, _L)] * e1
        a2 = n_v[pl.ds(2 * _C + j, _L)] * e2
        ox_v[pl.ds(ji, _L)] = (r00 * a0 + r01 * a1 + r02 * a2 + px) * m
        ox_v[pl.ds(ji + _B, _L)] = (r10 * a0 + r11 * a1 + r12 * a2 + py) * m
        ox_v[pl.ds(ji + 2 * _B, _L)] = (r20 * a0 + r21 * a1 + r22 * a2
                                        + pz) * m

      os_v[pl.ds(ji, _L)] = (s0 - _LOG_SPLIT) * m
      os_v[pl.ds(ji + _B, _L)] = (s1 - _LOG_SPLIT) * m
      os_v[pl.ds(ji + 2 * _B, _L)] = (s2 - _LOG_SPLIT) * m
      or_v[pl.ds(ji, _L)] = q0 * m
      or_v[pl.ds(ji + _B, _L)] = q1 * m
      or_v[pl.ds(ji + 2 * _B, _L)] = q2 * m
      or_v[pl.ds(ji + 3 * _B, _L)] = q3 * m
      return scarry

    lax.fori_loop(0, _C // _L, step, 0)

  pending_in = {0: start_in(0, 0)}
  pending_out = {}
  for k in range(_NCHUNK):
    b = k % _NBUF
    if k + 1 < _NCHUNK:
      pending_in[k + 1] = start_in(k + 1, (k + 1) % _NBUF)
    for h in pending_in.pop(k):
      h.wait()
    if k - _NBUF in pending_out:
      for h in pending_out.pop(k - _NBUF):
        h.wait()
    compute(b)
    pending_out[k] = start_out(k, b)
  for k in sorted(pending_out):
    for h in pending_out[k]:
      h.wait()


_densify = functools.partial(
    pl.kernel,
    out_type=(
        jax.ShapeDtypeStruct((4 * _M,), jnp.float32),
        jax.ShapeDtypeStruct((4 * _M,), jnp.float32),
        jax.ShapeDtypeStruct((4 * _M,), jnp.float32),
    ),
    mesh=plsc.VectorSubcoreMesh(core_axis_name="c", subcore_axis_name="s"),
    compiler_params=pltpu.CompilerParams(
        needs_layout_passes=False, use_tc_tiling_on_sc=False),
    scratch_types=(
        [t for _ in range(_NBUF) for t in (
            pltpu.VMEM((3 * _C,), jnp.float32),
            pltpu.VMEM((3 * _C,), jnp.float32),
            pltpu.VMEM((4 * _C,), jnp.float32),
            pltpu.VMEM((_C,), jnp.float32),
            pltpu.VMEM((3 * _C,), jnp.float32),
            pltpu.VMEM((3 * _C,), jnp.float32),
        )] +
        [t for _ in range(_NBUF) for t in (
            pltpu.VMEM((4 * _C,), jnp.float32),
            pltpu.VMEM((4 * _C,), jnp.float32),
            pltpu.VMEM((4 * _C,), jnp.float32),
            pltpu.VMEM((4 * _C,), jnp.float32),
        )] +
        [pltpu.SemaphoreType.DMA] * 4
    ),
)(_densify_body)


def kernel(xyz, scaling, rotation, grads, noise):
  rot_f = rotation.reshape(_N // _B, _B, 4).transpose(0, 2, 1).reshape(-1)
  fxyz, fscal, frot = _densify(
      xyz.T.reshape(-1), scaling.T.reshape(-1), rot_f,
      grads.reshape(-1), noise.T.reshape(-1))
  nxyz = fxyz.reshape(_M // _B, 4, _B)[:, :3, :].transpose(0, 2, 1)
  nscal = fscal.reshape(_M // _B, 4, _B)[:, :3, :].transpose(0, 2, 1)
  nrot = frot.reshape(_M // _B, 4, _B).transpose(0, 2, 1)
  return (nxyz.reshape(_M, 3), nscal.reshape(_M, 3), nrot.reshape(_M, 4))
